# baseline (speedup 1.0000x reference)
import functools

import jax
import jax.numpy as jnp
from jax import lax
from jax.experimental import pallas as pl
from jax.experimental.pallas import tpu as pltpu
import jax.experimental.pallas.tpu_sc as plsc

BN = 1000
NC, NS, L = 2, 16, 16
NW = NC * NS
CH = 80


def _mlp_kernel(x_ref, w1_ref, b1_ref, w2a_ref, b2a_ref, w2b_ref, b2b_ref,
                gf_ref, gw1_ref, gb1_ref, gw2_ref, gb2_ref,
                h2a_ref, h2b_ref, g_ref):
    x = x_ref[...]
    h = jnp.maximum(
        jnp.dot(x, w1_ref[...], preferred_element_type=jnp.float32)
        + b1_ref[...], 0.0)
    h2a_ref[...] = (jnp.dot(h, w2a_ref[...],
                            preferred_element_type=jnp.float32) + b2a_ref[...])
    h2b_ref[...] = (jnp.dot(h, w2b_ref[...],
                            preferred_element_type=jnp.float32) + b2b_ref[...])

    @pl.when(pl.program_id(0) == 0)
    def _():
        gh = jnp.maximum(
            jnp.dot(gf_ref[...], gw1_ref[...],
                    preferred_element_type=jnp.float32) + gb1_ref[...], 0.0)
        g_ref[...] = (jnp.dot(gh, gw2_ref[...],
                              preferred_element_type=jnp.float32)
                      + gb2_ref[...])


def _combine_kernel(x_ref, msga_ref, msgb_ref, den_ref, g_ref,
                    w1_ref, b1_ref, w2_ref, b2_ref, out_ref):
    x = x_ref[...]
    h = jnp.maximum(
        jnp.dot(x, w1_ref[...], preferred_element_type=jnp.float32)
        + b1_ref[...], 0.0)
    h1 = (jnp.dot(h, w2_ref[...], preferred_element_type=jnp.float32)
          + b2_ref[...])
    msg = jnp.concatenate([msga_ref[0] + msga_ref[1],
                           msgb_ref[0] + msgb_ref[1]], axis=1)
    den = jnp.sum(den_ref[...], axis=1)[:, None] + 1e-07
    inter = h1 + msg / den + g_ref[...]
    mean = jnp.mean(inter, axis=1, keepdims=True)
    var = jnp.mean((inter - mean) ** 2, axis=1, keepdims=True)
    normed = (inter - mean) * lax.rsqrt(var + 1e-05)
    out_ref[...] = x + jnp.maximum(normed, 0.0)


def _full_spec(shape):
    return pl.BlockSpec(shape, lambda i: (0,) * len(shape))


def _sc_edge_body(Np, dh, NCHUNK,
                  h2a_hbm, h2b_hbm, src_hbm, dst_hbm, ef_hbm,
                  msga_hbm, msgb_hbm, den_hbm,
                  accum_sh, srcw, dstw, sigw, rows0, rows1, denv,
                  sem_g0, sem_g1, sem_s0, sem_s1):
    c = lax.axis_index("c")
    s = lax.axis_index("s")
    wid = c * NS + s
    rows_per_s = Np // NS
    nwb = rows_per_s // CH

    pltpu.sync_copy(src_hbm.at[wid], srcw)
    pltpu.sync_copy(dst_hbm.at[wid], dstw)
    pltpu.sync_copy(ef_hbm.at[wid], sigw)

    z16 = jnp.zeros((L,), jnp.float32)

    def zero_den(i, _):
        denv[pl.ds(pl.multiple_of(i * L, L), L)] = z16
        return 0
    lax.fori_loop(0, Np // L, zero_den, 0)

    def sig_den(i, _):
        for k in range(CH // L):
            sl = pl.ds(k * L, L)
            sg = 1.0 / (1.0 + jnp.exp(-sigw[i, sl]))
            sigw[i, sl] = sg
            plsc.addupdate_scatter(denv, [srcw[i, sl]], sg)
            plsc.addupdate_scatter(denv, [dstw[i, sl]], sg)
        return 0
    lax.fori_loop(0, NCHUNK, sig_den, 0)

    def wait_dma(sem, buf):
        pltpu.make_async_copy(h2a_hbm.at[pl.ds(0, CH)], buf, sem).wait()

    def scale(rows, ci):
        def scale_group(gi, _):
            sg16 = sigw[ci, pl.ds(pl.multiple_of(gi * L, L), L)]
            rbase = gi * L
            for rr in range(L):
                sv = sg16[rr]
                for j in range(dh // L):
                    rows[rbase + rr, pl.ds(j * L, L)] = (
                        rows[rbase + rr, pl.ds(j * L, L)] * sv)
            return 0
        lax.fori_loop(0, CH // L, scale_group, 0)

    for h2h, msgh in ((h2a_hbm, msga_hbm), (h2b_hbm, msgb_hbm)):
        def zero_rows0(i, _):
            for j in range(dh // L):
                rows0[i, pl.ds(j * L, L)] = z16
            return 0
        lax.fori_loop(0, CH, zero_rows0, 0)
        for k in range(nwb):
            pltpu.sync_copy(rows0,
                            accum_sh.at[pl.ds(s * rows_per_s + k * CH, CH)])
        plsc.subcore_barrier()

        pltpu.async_copy(h2h.at[dstw.at[0]], rows0, sem_g0)
        pltpu.async_copy(h2h.at[srcw.at[0]], rows1, sem_g1)

        def chunk_body(ci, _):
            wait_dma(sem_g0, rows0)
            scale(rows0, ci)
            pltpu.async_copy(rows0, accum_sh.at[srcw.at[ci]], sem_s0,
                             add=True)
            wait_dma(sem_g1, rows1)
            scale(rows1, ci)
            pltpu.async_copy(rows1, accum_sh.at[dstw.at[ci]], sem_s1,
                             add=True)

            @pl.when(ci < NCHUNK - 1)
            def _():
                wait_dma(sem_s0, rows0)
                pltpu.async_copy(h2h.at[dstw.at[ci + 1]], rows0, sem_g0)
                wait_dma(sem_s1, rows1)
                pltpu.async_copy(h2h.at[srcw.at[ci + 1]], rows1, sem_g1)
            return 0

        lax.fori_loop(0, NCHUNK, chunk_body, 0)
        wait_dma(sem_s0, rows0)
        wait_dma(sem_s1, rows1)
        plsc.subcore_barrier()

        for k in range(nwb):
            start = s * rows_per_s + k * CH
            buf = rows0 if k % 2 == 0 else rows1
            pltpu.sync_copy(accum_sh.at[pl.ds(start, CH)], buf)
            pltpu.sync_copy(buf, msgh.at[c, pl.ds(start, CH)])

    pltpu.sync_copy(denv, den_hbm.at[wid])


def _sc_edge(h2a, h2b, src, dst, ef):
    N, dh = h2a.shape
    E = src.shape[0]
    EW = E // NW
    NCHUNK = EW // CH
    Np = -(-N // (NS * 128)) * (NS * 128)
    mesh = plsc.VectorSubcoreMesh(core_axis_name="c", subcore_axis_name="s")
    f = pl.kernel(
        functools.partial(_sc_edge_body, Np, dh, NCHUNK),
        out_type=(jax.ShapeDtypeStruct((NC, Np, dh), jnp.float32),
                  jax.ShapeDtypeStruct((NC, Np, dh), jnp.float32),
                  jax.ShapeDtypeStruct((NW, Np), jnp.float32)),
        mesh=mesh,
        scratch_types=[
            pltpu.VMEM_SHARED((Np, dh), jnp.float32),
            pltpu.VMEM((NCHUNK, CH), jnp.int32),
            pltpu.VMEM((NCHUNK, CH), jnp.int32),
            pltpu.VMEM((NCHUNK, CH), jnp.float32),
            pltpu.VMEM((CH, dh), jnp.float32),
            pltpu.VMEM((CH, dh), jnp.float32),
            pltpu.VMEM((Np,), jnp.float32),
            pltpu.SemaphoreType.DMA,
            pltpu.SemaphoreType.DMA,
            pltpu.SemaphoreType.DMA,
            pltpu.SemaphoreType.DMA,
        ],
        compiler_params=pltpu.CompilerParams(needs_layout_passes=False,
                                             use_tc_tiling_on_sc=False),
    )
    msga, msgb, den = f(h2a, h2b,
                        src.reshape(NW, NCHUNK, CH),
                        dst.reshape(NW, NCHUNK, CH),
                        ef.reshape(NW, NCHUNK, CH))
    return msga[:, :N], msgb[:, :N], den[:, :N]


def kernel(node_features, edge_index, edge_features, global_features,
           W1a, b1a, W2a, b2a, W1b, b1b, W2b, b2b, W1c, b1c, W2c, b2c):
    x = node_features[0]
    N, d = x.shape
    hdim = W1a.shape[0]
    dh = d // 2
    src = edge_index[0, 0]
    dst = edge_index[0, 1]

    grid = N // BN
    row_spec = pl.BlockSpec((BN, d), lambda i: (i, 0))
    half_spec = pl.BlockSpec((BN, dh), lambda i: (i, 0))

    W2bT = W2b.T
    h2a, h2b, g = pl.pallas_call(
        _mlp_kernel,
        grid=(grid,),
        in_specs=[
            row_spec,
            _full_spec((d, hdim)), _full_spec((1, hdim)),
            _full_spec((hdim, dh)), _full_spec((1, dh)),
            _full_spec((hdim, dh)), _full_spec((1, dh)),
            _full_spec((1, d)),
            _full_spec((d, hdim)), _full_spec((1, hdim)),
            _full_spec((hdim, d)), _full_spec((1, d)),
        ],
        out_specs=[half_spec, half_spec, _full_spec((1, d))],
        out_shape=[jax.ShapeDtypeStruct((N, dh), jnp.float32),
                   jax.ShapeDtypeStruct((N, dh), jnp.float32),
                   jax.ShapeDtypeStruct((1, d), jnp.float32)],
    )(x, W1b.T, b1b[None], W2bT[:, :dh], b2b[None, :dh],
      W2bT[:, dh:], b2b[None, dh:],
      global_features[0], W1c.T, b1c[None], W2c.T, b2c[None])

    msga, msgb, den32 = _sc_edge(h2a, h2b, src, dst, edge_features[0])

    out = pl.pallas_call(
        _combine_kernel,
        grid=(grid,),
        in_specs=[
            row_spec,
            pl.BlockSpec((NC, BN, dh), lambda i: (0, i, 0)),
            pl.BlockSpec((NC, BN, dh), lambda i: (0, i, 0)),
            pl.BlockSpec((BN, NW), lambda i: (i, 0)),
            _full_spec((1, d)),
            _full_spec((d, hdim)), _full_spec((1, hdim)),
            _full_spec((hdim, d)), _full_spec((1, d)),
        ],
        out_specs=row_spec,
        out_shape=jax.ShapeDtypeStruct((N, d), jnp.float32),
    )(x, msga, msgb, den32.T, g,
      W1a.T, b1a[None], W2a.T, b2a[None])

    return out[None]


# speedup vs baseline: 1.1619x; 1.1619x over previous
import functools

import jax
import jax.numpy as jnp
from jax import lax
from jax.experimental import pallas as pl
from jax.experimental.pallas import tpu as pltpu
import jax.experimental.pallas.tpu_sc as plsc

BN = 1000
NC, NS, L = 2, 16, 16
NW = NC * NS
CH = 80


def _mlp_kernel(x_ref, w1_ref, b1_ref, w2a_ref, b2a_ref, w2b_ref, b2b_ref,
                gf_ref, gw1_ref, gb1_ref, gw2_ref, gb2_ref,
                h2a_ref, h2b_ref, g_ref):
    x = x_ref[...]
    h = jnp.maximum(
        jnp.dot(x, w1_ref[...], preferred_element_type=jnp.float32)
        + b1_ref[...], 0.0)
    h2a_ref[...] = (jnp.dot(h, w2a_ref[...],
                            preferred_element_type=jnp.float32) + b2a_ref[...])
    h2b_ref[...] = (jnp.dot(h, w2b_ref[...],
                            preferred_element_type=jnp.float32) + b2b_ref[...])

    @pl.when(pl.program_id(0) == 0)
    def _():
        gh = jnp.maximum(
            jnp.dot(gf_ref[...], gw1_ref[...],
                    preferred_element_type=jnp.float32) + gb1_ref[...], 0.0)
        g_ref[...] = (jnp.dot(gh, gw2_ref[...],
                              preferred_element_type=jnp.float32)
                      + gb2_ref[...])


def _combine_kernel(x_ref, msga_ref, msgb_ref, den_ref, g_ref,
                    w1_ref, b1_ref, w2_ref, b2_ref, out_ref):
    x = x_ref[...]
    h = jnp.maximum(
        jnp.dot(x, w1_ref[...], preferred_element_type=jnp.float32)
        + b1_ref[...], 0.0)
    h1 = (jnp.dot(h, w2_ref[...], preferred_element_type=jnp.float32)
          + b2_ref[...])
    msg = jnp.concatenate([msga_ref[0] + msga_ref[1],
                           msgb_ref[0] + msgb_ref[1]], axis=1)
    den = jnp.sum(den_ref[...], axis=1)[:, None] + 1e-07
    inter = h1 + msg / den + g_ref[...]
    mean = jnp.mean(inter, axis=1, keepdims=True)
    var = jnp.mean((inter - mean) ** 2, axis=1, keepdims=True)
    normed = (inter - mean) * lax.rsqrt(var + 1e-05)
    out_ref[...] = x + jnp.maximum(normed, 0.0)


def _full_spec(shape):
    return pl.BlockSpec(shape, lambda i: (0,) * len(shape))


def _sc_edge_body(Np, dh, NCHUNK,
                  h2a_hbm, h2b_hbm, src_hbm, dst_hbm, ef_hbm,
                  msga_hbm, msgb_hbm, den_hbm,
                  accum_sh, srcw, dstw, sigw, rows0a, rows0b, rows1a, rows1b,
                  denv, sem_g0, sem_g1, sem_s0, sem_s1):
    c = lax.axis_index("c")
    s = lax.axis_index("s")
    wid = c * NS + s
    rows_per_s = Np // NS
    nwb = rows_per_s // CH

    pltpu.sync_copy(src_hbm.at[wid], srcw)
    pltpu.sync_copy(dst_hbm.at[wid], dstw)
    pltpu.sync_copy(ef_hbm.at[wid], sigw)

    z16 = jnp.zeros((L,), jnp.float32)

    def zero_den(i, _):
        denv[pl.ds(pl.multiple_of(i * L, L), L)] = z16
        return 0
    lax.fori_loop(0, Np // L, zero_den, 0)

    def sig_den(i, _):
        for k in range(CH // L):
            sl = pl.ds(k * L, L)
            sg = 1.0 / (1.0 + jnp.exp(-sigw[i, sl]))
            sigw[i, sl] = sg
            plsc.addupdate_scatter(denv, [srcw[i, sl]], sg)
            plsc.addupdate_scatter(denv, [dstw[i, sl]], sg)
        return 0
    lax.fori_loop(0, NCHUNK, sig_den, 0)

    def wait_dma(sem, buf):
        pltpu.make_async_copy(h2a_hbm.at[pl.ds(0, CH)], buf, sem).wait()

    def scale(rows, ci):
        def scale_group(gi, _):
            sg16 = sigw[ci, pl.ds(pl.multiple_of(gi * L, L), L)]
            rbase = gi * L
            for rr in range(L):
                sv = sg16[rr]
                for j in range(dh // L):
                    rows[rbase + rr, pl.ds(j * L, L)] = (
                        rows[rbase + rr, pl.ds(j * L, L)] * sv)
            return 0
        lax.fori_loop(0, CH // L, scale_group, 0)

    for h2h, msgh in ((h2a_hbm, msga_hbm), (h2b_hbm, msgb_hbm)):
        def zero_rows0(i, _):
            for j in range(dh // L):
                rows0a[i, pl.ds(j * L, L)] = z16
            return 0
        lax.fori_loop(0, CH, zero_rows0, 0)
        for k in range(nwb):
            pltpu.sync_copy(rows0a,
                            accum_sh.at[pl.ds(s * rows_per_s + k * CH, CH)])
        plsc.subcore_barrier()

        def gathers(ci, b0, b1):
            pltpu.async_copy(h2h.at[dstw.at[ci]], b0, sem_g0)
            pltpu.async_copy(h2h.at[srcw.at[ci]], b1, sem_g1)

        def compute_chunk(ci, b0, b1):
            wait_dma(sem_g0, b0)
            scale(b0, ci)
            pltpu.async_copy(b0, accum_sh.at[srcw.at[ci]], sem_s0, add=True)
            wait_dma(sem_g1, b1)
            scale(b1, ci)
            pltpu.async_copy(b1, accum_sh.at[dstw.at[ci]], sem_s1, add=True)

        def chunk_step(ci, p0, p1, q0, q1, prefetch):
            if prefetch is not None:
                wait_dma(sem_s0, q0)
                wait_dma(sem_s1, q1)
                if prefetch is True:
                    gathers(ci + 1, q0, q1)
                else:

                    @pl.when(prefetch)
                    def _():
                        gathers(ci + 1, q0, q1)
            compute_chunk(ci, p0, p1)

        gathers(0, rows0a, rows1a)
        gathers(1, rows0b, rows1b)
        compute_chunk(0, rows0a, rows1a)

        def pair_body(t, _):
            c1 = 2 * t + 1
            chunk_step(c1, rows0b, rows1b, rows0a, rows1a, True)
            chunk_step(c1 + 1, rows0a, rows1a, rows0b, rows1b,
                       c1 + 2 < NCHUNK)
            return 0

        lax.fori_loop(0, (NCHUNK - 1) // 2, pair_body, 0)
        wait_dma(sem_s0, rows0a)
        wait_dma(sem_s1, rows1a)
        plsc.subcore_barrier()

        for k in range(nwb):
            start = s * rows_per_s + k * CH
            buf = rows0a if k % 2 == 0 else rows0b
            pltpu.sync_copy(accum_sh.at[pl.ds(start, CH)], buf)
            pltpu.sync_copy(buf, msgh.at[c, pl.ds(start, CH)])

    pltpu.sync_copy(denv, den_hbm.at[wid])


def _sc_edge(h2a, h2b, src, dst, ef):
    N, dh = h2a.shape
    E = src.shape[0]
    EW = E // NW
    NCHUNK = EW // CH
    Np = -(-N // (NS * 128)) * (NS * 128)
    mesh = plsc.VectorSubcoreMesh(core_axis_name="c", subcore_axis_name="s")
    f = pl.kernel(
        functools.partial(_sc_edge_body, Np, dh, NCHUNK),
        out_type=(jax.ShapeDtypeStruct((NC, Np, dh), jnp.float32),
                  jax.ShapeDtypeStruct((NC, Np, dh), jnp.float32),
                  jax.ShapeDtypeStruct((NW, Np), jnp.float32)),
        mesh=mesh,
        scratch_types=[
            pltpu.VMEM_SHARED((Np, dh), jnp.float32),
            pltpu.VMEM((NCHUNK, CH), jnp.int32),
            pltpu.VMEM((NCHUNK, CH), jnp.int32),
            pltpu.VMEM((NCHUNK, CH), jnp.float32),
            pltpu.VMEM((CH, dh), jnp.float32),
            pltpu.VMEM((CH, dh), jnp.float32),
            pltpu.VMEM((CH, dh), jnp.float32),
            pltpu.VMEM((CH, dh), jnp.float32),
            pltpu.VMEM((Np,), jnp.float32),
            pltpu.SemaphoreType.DMA,
            pltpu.SemaphoreType.DMA,
            pltpu.SemaphoreType.DMA,
            pltpu.SemaphoreType.DMA,
        ],
        compiler_params=pltpu.CompilerParams(needs_layout_passes=False,
                                             use_tc_tiling_on_sc=False),
    )
    msga, msgb, den = f(h2a, h2b,
                        src.reshape(NW, NCHUNK, CH),
                        dst.reshape(NW, NCHUNK, CH),
                        ef.reshape(NW, NCHUNK, CH))
    return msga[:, :N], msgb[:, :N], den[:, :N]


def kernel(node_features, edge_index, edge_features, global_features,
           W1a, b1a, W2a, b2a, W1b, b1b, W2b, b2b, W1c, b1c, W2c, b2c):
    x = node_features[0]
    N, d = x.shape
    hdim = W1a.shape[0]
    dh = d // 2
    src = edge_index[0, 0]
    dst = edge_index[0, 1]

    grid = N // BN
    row_spec = pl.BlockSpec((BN, d), lambda i: (i, 0))
    half_spec = pl.BlockSpec((BN, dh), lambda i: (i, 0))

    W2bT = W2b.T
    h2a, h2b, g = pl.pallas_call(
        _mlp_kernel,
        grid=(grid,),
        in_specs=[
            row_spec,
            _full_spec((d, hdim)), _full_spec((1, hdim)),
            _full_spec((hdim, dh)), _full_spec((1, dh)),
            _full_spec((hdim, dh)), _full_spec((1, dh)),
            _full_spec((1, d)),
            _full_spec((d, hdim)), _full_spec((1, hdim)),
            _full_spec((hdim, d)), _full_spec((1, d)),
        ],
        out_specs=[half_spec, half_spec, _full_spec((1, d))],
        out_shape=[jax.ShapeDtypeStruct((N, dh), jnp.float32),
                   jax.ShapeDtypeStruct((N, dh), jnp.float32),
                   jax.ShapeDtypeStruct((1, d), jnp.float32)],
    )(x, W1b.T, b1b[None], W2bT[:, :dh], b2b[None, :dh],
      W2bT[:, dh:], b2b[None, dh:],
      global_features[0], W1c.T, b1c[None], W2c.T, b2c[None])

    msga, msgb, den32 = _sc_edge(h2a, h2b, src, dst, edge_features[0])

    out = pl.pallas_call(
        _combine_kernel,
        grid=(grid,),
        in_specs=[
            row_spec,
            pl.BlockSpec((NC, BN, dh), lambda i: (0, i, 0)),
            pl.BlockSpec((NC, BN, dh), lambda i: (0, i, 0)),
            pl.BlockSpec((BN, NW), lambda i: (i, 0)),
            _full_spec((1, d)),
            _full_spec((d, hdim)), _full_spec((1, hdim)),
            _full_spec((hdim, d)), _full_spec((1, d)),
        ],
        out_specs=row_spec,
        out_shape=jax.ShapeDtypeStruct((N, d), jnp.float32),
    )(x, msga, msgb, den32.T, g,
      W1a.T, b1a[None], W2a.T, b2a[None])

    return out[None]


# speedup vs baseline: 2.2072x; 1.8997x over previous
import functools

import jax
import jax.numpy as jnp
from jax import lax
from jax.experimental import pallas as pl
from jax.experimental.pallas import tpu as pltpu
import jax.experimental.pallas.tpu_sc as plsc

BN = 1000
NC, NS, L = 2, 16, 16
NW = NC * NS
CH = 80


def _mlp_kernel(x_ref, w1_ref, b1_ref, w2_ref, b2_ref,
                gf_ref, gw1_ref, gb1_ref, gw2_ref, gb2_ref,
                h2_ref, g_ref):
    x = x_ref[...]
    h = jnp.maximum(
        jnp.dot(x, w1_ref[...], preferred_element_type=jnp.float32)
        + b1_ref[...], 0.0)
    h2_ref[...] = (jnp.dot(h, w2_ref[...], preferred_element_type=jnp.float32)
                   + b2_ref[...])

    @pl.when(pl.program_id(0) == 0)
    def _():
        gh = jnp.maximum(
            jnp.dot(gf_ref[...], gw1_ref[...],
                    preferred_element_type=jnp.float32) + gb1_ref[...], 0.0)
        g_ref[...] = (jnp.dot(gh, gw2_ref[...],
                              preferred_element_type=jnp.float32)
                      + gb2_ref[...])


def _combine_kernel(x_ref, msg_ref, den_ref, g_ref,
                    w1_ref, b1_ref, w2_ref, b2_ref, out_ref):
    x = x_ref[...]
    h = jnp.maximum(
        jnp.dot(x, w1_ref[...], preferred_element_type=jnp.float32)
        + b1_ref[...], 0.0)
    h1 = (jnp.dot(h, w2_ref[...], preferred_element_type=jnp.float32)
          + b2_ref[...])
    msg = msg_ref[0] + msg_ref[1]
    den = jnp.sum(den_ref[...], axis=1)[:, None] + 1e-07
    inter = h1 + msg / den + g_ref[...]
    mean = jnp.mean(inter, axis=1, keepdims=True)
    var = jnp.mean((inter - mean) ** 2, axis=1, keepdims=True)
    normed = (inter - mean) * lax.rsqrt(var + 1e-05)
    out_ref[...] = x + jnp.maximum(normed, 0.0)


def _full_spec(shape):
    return pl.BlockSpec(shape, lambda i: (0,) * len(shape))


def _sc_edge_body(Np, N, d, NCH,
                  h2_hbm, epk_hbm, msg_hbm, den_hbm,
                  accum_sh, eba, ebb, sidxa, sidxb, siga, sigb,
                  rowsa, rowsb, denv, sem_e, sem_g, sem_s):
    c = lax.axis_index("c")
    s = lax.axis_index("s")
    wid = c * NS + s
    rows_per_s = Np // NS
    nwb = rows_per_s // CH

    z16 = jnp.zeros((L,), jnp.float32)

    def zero_den(i, _):
        denv[pl.ds(pl.multiple_of(i * L, L), L)] = z16
        return 0
    lax.fori_loop(0, N // L, zero_den, 0)

    def zero_rows(i, _):
        for j in range(d // L):
            rowsa[i, pl.ds(j * L, L)] = z16
        return 0
    lax.fori_loop(0, CH, zero_rows, 0)
    for k in range(nwb):
        pltpu.sync_copy(rowsa, accum_sh.at[pl.ds(s * rows_per_s + k * CH, CH)])
    plsc.subcore_barrier()

    def wait_rows(sem):
        pltpu.make_async_copy(h2_hbm.at[pl.ds(0, CH)], rowsa, sem).wait()

    def wait_eb(sem):
        pltpu.make_async_copy(epk_hbm.at[0, 0], eba, sem).wait()

    def issue_e(ci, eb):
        pltpu.async_copy(epk_hbm.at[wid, ci], eb, sem_e)

    def issue_g(ci, eb, rows):
        pltpu.async_copy(h2_hbm.at[eb.at[1]], rows, sem_g)

    def consume_eb(eb, sidx, sig):
        for k in range(CH // L):
            sl = pl.ds(k * L, L)
            srcv = eb[0, sl]
            sidx[sl] = srcv
            efv = lax.bitcast_convert_type(eb[2, sl], jnp.float32)
            sg = 1.0 / (1.0 + jnp.exp(-efv))
            sig[sl] = sg
            plsc.addupdate_scatter(denv, [srcv], sg)

    def scale(rows, sig):
        def scale_group(gi, _):
            sg16 = sig[pl.ds(pl.multiple_of(gi * L, L), L)]
            rbase = gi * L
            for rr in range(L):
                sv = sg16[rr]
                for j in range(d // L):
                    rows[rbase + rr, pl.ds(j * L, L)] = (
                        rows[rbase + rr, pl.ds(j * L, L)] * sv)
            return 0
        lax.fori_loop(0, CH // L, scale_group, 0)

    def step(ci, ebp, sidxp, sigp, rowsp, ebq, rowsq):
        wait_rows(sem_g)
        consume_eb(ebp, sidxp, sigp)

        @pl.when(ci + 2 < NCH)
        def _():
            issue_e(ci + 2, ebp)

        @pl.when(ci > 0)
        def _():
            wait_rows(sem_s)

        @pl.when(ci + 1 < NCH)
        def _():
            wait_eb(sem_e)
            issue_g(ci + 1, ebq, rowsq)

        scale(rowsp, sigp)
        pltpu.async_copy(rowsp, accum_sh.at[sidxp], sem_s, add=True)

    issue_e(0, eba)
    wait_eb(sem_e)
    issue_g(0, eba, rowsa)
    issue_e(1, ebb)

    def pair_body(t, _):
        c1 = 2 * t
        step(c1, eba, sidxa, siga, rowsa, ebb, rowsb)
        step(c1 + 1, ebb, sidxb, sigb, rowsb, eba, rowsa)
        return 0

    lax.fori_loop(0, NCH // 2, pair_body, 0)
    wait_rows(sem_s)
    plsc.subcore_barrier()

    for k in range(nwb):
        start = s * rows_per_s + k * CH
        buf = rowsa if k % 2 == 0 else rowsb
        pltpu.sync_copy(accum_sh.at[pl.ds(start, CH)], buf)
        pltpu.sync_copy(buf, msg_hbm.at[c, pl.ds(start, CH)])
    pltpu.sync_copy(denv, den_hbm.at[wid])


def _sc_edge(h2, epk, N):
    d = h2.shape[1]
    NCH = epk.shape[1]
    Np = -(-N // (NS * 128)) * (NS * 128)
    mesh = plsc.VectorSubcoreMesh(core_axis_name="c", subcore_axis_name="s")
    f = pl.kernel(
        functools.partial(_sc_edge_body, Np, N, d, NCH),
        out_type=(jax.ShapeDtypeStruct((NC, Np, d), jnp.float32),
                  jax.ShapeDtypeStruct((NW, N), jnp.float32)),
        mesh=mesh,
        scratch_types=[
            pltpu.VMEM_SHARED((Np, d), jnp.float32),
            pltpu.VMEM((3, CH), jnp.int32),
            pltpu.VMEM((3, CH), jnp.int32),
            pltpu.VMEM((CH,), jnp.int32),
            pltpu.VMEM((CH,), jnp.int32),
            pltpu.VMEM((CH,), jnp.float32),
            pltpu.VMEM((CH,), jnp.float32),
            pltpu.VMEM((CH, d), jnp.float32),
            pltpu.VMEM((CH, d), jnp.float32),
            pltpu.VMEM((N,), jnp.float32),
            pltpu.SemaphoreType.DMA,
            pltpu.SemaphoreType.DMA,
            pltpu.SemaphoreType.DMA,
        ],
        compiler_params=pltpu.CompilerParams(needs_layout_passes=False,
                                             use_tc_tiling_on_sc=False),
    )
    msg2, den = f(h2, epk)
    return msg2[:, :N], den


def kernel(node_features, edge_index, edge_features, global_features,
           W1a, b1a, W2a, b2a, W1b, b1b, W2b, b2b, W1c, b1c, W2c, b2c):
    x = node_features[0]
    N, d = x.shape
    hdim = W1a.shape[0]
    src = edge_index[0, 0]
    dst = edge_index[0, 1]
    E = src.shape[0]
    NCH = 2 * E // (NW * CH)

    grid = N // BN
    row_spec = pl.BlockSpec((BN, d), lambda i: (i, 0))

    h2, g = pl.pallas_call(
        _mlp_kernel,
        grid=(grid,),
        in_specs=[
            row_spec,
            _full_spec((d, hdim)), _full_spec((1, hdim)),
            _full_spec((hdim, d)), _full_spec((1, d)),
            _full_spec((1, d)),
            _full_spec((d, hdim)), _full_spec((1, hdim)),
            _full_spec((hdim, d)), _full_spec((1, d)),
        ],
        out_specs=[row_spec, _full_spec((1, d))],
        out_shape=[jax.ShapeDtypeStruct((N, d), jnp.float32),
                   jax.ShapeDtypeStruct((1, d), jnp.float32)],
    )(x, W1b.T, b1b[None], W2b.T, b2b[None],
      global_features[0], W1c.T, b1c[None], W2c.T, b2c[None])

    ebits = lax.bitcast_convert_type(edge_features[0], jnp.int32)
    epk = jnp.stack([
        jnp.concatenate([src, dst]).reshape(NW, NCH, CH),
        jnp.concatenate([dst, src]).reshape(NW, NCH, CH),
        jnp.concatenate([ebits, ebits]).reshape(NW, NCH, CH),
    ], axis=2)

    msg2, den32 = _sc_edge(h2, epk, N)

    out = pl.pallas_call(
        _combine_kernel,
        grid=(grid,),
        in_specs=[
            row_spec,
            pl.BlockSpec((NC, BN, d), lambda i: (0, i, 0)),
            pl.BlockSpec((BN, NW), lambda i: (i, 0)),
            _full_spec((1, d)),
            _full_spec((d, hdim)), _full_spec((1, hdim)),
            _full_spec((hdim, d)), _full_spec((1, d)),
        ],
        out_specs=row_spec,
        out_shape=jax.ShapeDtypeStruct((N, d), jnp.float32),
    )(x, msg2, den32.T, g,
      W1a.T, b1a[None], W2a.T, b2a[None])

    return out[None]


# speedup vs baseline: 2.2337x; 1.0120x over previous
import functools

import jax
import jax.numpy as jnp
from jax import lax
from jax.experimental import pallas as pl
from jax.experimental.pallas import tpu as pltpu
import jax.experimental.pallas.tpu_sc as plsc

BN = 1000
NC, NS, L = 2, 16, 16
NW = NC * NS
CH = 80


def _mlp_kernel(x_ref, w1_ref, b1_ref, w2_ref, b2_ref,
                gf_ref, gw1_ref, gb1_ref, gw2_ref, gb2_ref,
                h2_ref, g_ref):
    x = x_ref[...]
    h = jnp.maximum(
        jnp.dot(x, w1_ref[...], preferred_element_type=jnp.float32)
        + b1_ref[...], 0.0)
    h2_ref[...] = (jnp.dot(h, w2_ref[...], preferred_element_type=jnp.float32)
                   + b2_ref[...])

    @pl.when(pl.program_id(0) == 0)
    def _():
        gh = jnp.maximum(
            jnp.dot(gf_ref[...], gw1_ref[...],
                    preferred_element_type=jnp.float32) + gb1_ref[...], 0.0)
        g_ref[...] = (jnp.dot(gh, gw2_ref[...],
                              preferred_element_type=jnp.float32)
                      + gb2_ref[...])


def _combine_kernel(x_ref, msg_ref, den_ref, g_ref,
                    w1_ref, b1_ref, w2_ref, b2_ref, out_ref):
    x = x_ref[...]
    h = jnp.maximum(
        jnp.dot(x, w1_ref[...], preferred_element_type=jnp.float32)
        + b1_ref[...], 0.0)
    h1 = (jnp.dot(h, w2_ref[...], preferred_element_type=jnp.float32)
          + b2_ref[...])
    msg = msg_ref[0] + msg_ref[1]
    den = jnp.sum(den_ref[...], axis=1)[:, None] + 1e-07
    inter = h1 + msg / den + g_ref[...]
    mean = jnp.mean(inter, axis=1, keepdims=True)
    var = jnp.mean((inter - mean) ** 2, axis=1, keepdims=True)
    normed = (inter - mean) * lax.rsqrt(var + 1e-05)
    out_ref[...] = x + jnp.maximum(normed, 0.0)


def _full_spec(shape):
    return pl.BlockSpec(shape, lambda i: (0,) * len(shape))


def _sc_edge_body(Np, N, d, NCH,
                  h2_hbm, epk_hbm, msg_hbm, den_hbm,
                  accum_sh, eba, ebb, sidxa, sidxb, siga, sigb,
                  rowsa, rowsb, denv, sem_e, sem_g, sem_s):
    c = lax.axis_index("c")
    s = lax.axis_index("s")
    wid = c * NS + s
    rows_per_s = Np // NS
    nwb = rows_per_s // CH

    z16 = jnp.zeros((L,), jnp.float32)

    def zero_den(i, _):
        denv[pl.ds(pl.multiple_of(i * L, L), L)] = z16
        return 0
    lax.fori_loop(0, N // L, zero_den, 0)

    def zero_rows(i, _):
        for j in range(d // L):
            rowsa[i, pl.ds(j * L, L)] = z16
        return 0
    lax.fori_loop(0, CH, zero_rows, 0)
    for k in range(nwb):
        pltpu.sync_copy(rowsa, accum_sh.at[pl.ds(s * rows_per_s + k * CH, CH)])
    plsc.subcore_barrier()

    def wait_rows(sem):
        pltpu.make_async_copy(h2_hbm.at[pl.ds(0, CH)], rowsa, sem).wait()

    def wait_eb(sem):
        pltpu.make_async_copy(epk_hbm.at[0, 0], eba, sem).wait()

    def issue_e(ci, eb):
        pltpu.async_copy(epk_hbm.at[wid, ci], eb, sem_e)

    def issue_g(ci, eb, rows):
        pltpu.async_copy(h2_hbm.at[eb.at[1]], rows, sem_g)

    def consume_eb(eb, sidx, sig):
        for k in range(CH // L):
            sl = pl.ds(k * L, L)
            srcv = eb[0, sl]
            sidx[sl] = srcv
            efv = lax.bitcast_convert_type(eb[2, sl], jnp.float32)
            sg = 1.0 / (1.0 + jnp.exp(-efv))
            sig[sl] = sg
            plsc.addupdate_scatter(denv, [srcv], sg)

    def scale(rows, sig):
        def scale_group(gi, _):
            sg16 = sig[pl.ds(pl.multiple_of(gi * L, L), L)]
            rbase = gi * L
            for rr in range(L):
                sv = sg16[rr]
                for j in range(d // L):
                    rows[rbase + rr, pl.ds(j * L, L)] = (
                        rows[rbase + rr, pl.ds(j * L, L)] * sv)
            return 0
        lax.fori_loop(0, CH // L, scale_group, 0)

    def step(ci, ebp, sidxp, sigp, rowsp, ebq, rowsq):
        wait_rows(sem_g)
        consume_eb(ebp, sidxp, sigp)

        @pl.when(ci + 2 < NCH)
        def _():
            issue_e(ci + 2, ebp)

        @pl.when(ci > 0)
        def _():
            wait_rows(sem_s)

        @pl.when(ci + 1 < NCH)
        def _():
            wait_eb(sem_e)
            issue_g(ci + 1, ebq, rowsq)

        scale(rowsp, sigp)
        pltpu.async_copy(rowsp, accum_sh.at[sidxp], sem_s, add=True)

    issue_e(0, eba)
    wait_eb(sem_e)
    issue_g(0, eba, rowsa)
    issue_e(1, ebb)

    def pair_body(t, _):
        c1 = 2 * t
        step(c1, eba, sidxa, siga, rowsa, ebb, rowsb)
        step(c1 + 1, ebb, sidxb, sigb, rowsb, eba, rowsa)
        return 0

    lax.fori_loop(0, NCH // 2, pair_body, 0)
    wait_rows(sem_s)
    plsc.subcore_barrier()

    for k in range(nwb):
        start = s * rows_per_s + k * CH
        buf = rowsa if k % 2 == 0 else rowsb
        pltpu.sync_copy(accum_sh.at[pl.ds(start, CH)], buf)
        pltpu.sync_copy(buf, msg_hbm.at[c, pl.ds(start, CH)])
    pltpu.sync_copy(denv, den_hbm.at[wid])


def _sc_edge(h2, epk, N):
    d = h2.shape[1]
    NCH = epk.shape[1]
    Np = -(-N // (NS * 128)) * (NS * 128)
    mesh = plsc.VectorSubcoreMesh(core_axis_name="c", subcore_axis_name="s")
    f = pl.kernel(
        functools.partial(_sc_edge_body, Np, N, d, NCH),
        out_type=(jax.ShapeDtypeStruct((NC, Np, d), jnp.float32),
                  jax.ShapeDtypeStruct((NW, N), jnp.float32)),
        mesh=mesh,
        scratch_types=[
            pltpu.VMEM_SHARED((Np, d), jnp.float32),
            pltpu.VMEM((3, CH), jnp.int32),
            pltpu.VMEM((3, CH), jnp.int32),
            pltpu.VMEM((CH,), jnp.int32),
            pltpu.VMEM((CH,), jnp.int32),
            pltpu.VMEM((CH,), jnp.float32),
            pltpu.VMEM((CH,), jnp.float32),
            pltpu.VMEM((CH, d), jnp.float32),
            pltpu.VMEM((CH, d), jnp.float32),
            pltpu.VMEM((N,), jnp.float32),
            pltpu.SemaphoreType.DMA,
            pltpu.SemaphoreType.DMA,
            pltpu.SemaphoreType.DMA,
        ],
        compiler_params=pltpu.CompilerParams(needs_layout_passes=False,
                                             use_tc_tiling_on_sc=False),
    )
    return f(h2, epk)


def kernel(node_features, edge_index, edge_features, global_features,
           W1a, b1a, W2a, b2a, W1b, b1b, W2b, b2b, W1c, b1c, W2c, b2c):
    x = node_features[0]
    N, d = x.shape
    hdim = W1a.shape[0]
    src = edge_index[0, 0]
    dst = edge_index[0, 1]
    E = src.shape[0]
    NCH = 2 * E // (NW * CH)

    grid = N // BN
    row_spec = pl.BlockSpec((BN, d), lambda i: (i, 0))

    h2, g = pl.pallas_call(
        _mlp_kernel,
        grid=(grid,),
        in_specs=[
            row_spec,
            _full_spec((d, hdim)), _full_spec((1, hdim)),
            _full_spec((hdim, d)), _full_spec((1, d)),
            _full_spec((1, d)),
            _full_spec((d, hdim)), _full_spec((1, hdim)),
            _full_spec((hdim, d)), _full_spec((1, d)),
        ],
        out_specs=[row_spec, _full_spec((1, d))],
        out_shape=[jax.ShapeDtypeStruct((N, d), jnp.float32),
                   jax.ShapeDtypeStruct((1, d), jnp.float32)],
    )(x, W1b.T, b1b[None], W2b.T, b2b[None],
      global_features[0], W1c.T, b1c[None], W2c.T, b2c[None])

    ebits = lax.bitcast_convert_type(edge_features[0], jnp.int32)
    epk = jnp.stack([
        jnp.concatenate([src, dst]).reshape(NW, NCH, CH),
        jnp.concatenate([dst, src]).reshape(NW, NCH, CH),
        jnp.concatenate([ebits, ebits]).reshape(NW, NCH, CH),
    ], axis=2)

    msg2, den32 = _sc_edge(h2, epk, N)

    out = pl.pallas_call(
        _combine_kernel,
        grid=(grid,),
        in_specs=[
            row_spec,
            pl.BlockSpec((NC, BN, d), lambda i: (0, i, 0)),
            pl.BlockSpec((BN, NW), lambda i: (i, 0)),
            _full_spec((1, d)),
            _full_spec((d, hdim)), _full_spec((1, hdim)),
            _full_spec((hdim, d)), _full_spec((1, d)),
        ],
        out_specs=row_spec,
        out_shape=jax.ShapeDtypeStruct((N, d), jnp.float32),
    )(x, msg2, den32.T, g,
      W1a.T, b1a[None], W2a.T, b2a[None])

    return out[None]
